# baseline (speedup 1.0000x reference)
import functools

import jax
import jax.numpy as jnp
from jax.experimental import pallas as pl
from jax.experimental.pallas import tpu as pltpu


def _split3(a):
    a1 = a.astype(jnp.bfloat16)
    r = a - a1.astype(jnp.float32)
    a2 = r.astype(jnp.bfloat16)
    r2 = r - a2.astype(jnp.float32)
    a3 = r2.astype(jnp.bfloat16)
    return a1, a2, a3


def _split2(a):
    a1 = a.astype(jnp.bfloat16)
    a2 = (a - a1.astype(jnp.float32)).astype(jnp.bfloat16)
    return a1, a2


def _dot(a, b):
    return jax.lax.dot_general(
        a, b, (((1,), (0,)), ((), ())),
        preferred_element_type=jnp.float32,
    )


def _tm_kernel(side, std_ref, x_ref, w_ref, out_ref, wc_ref, wg_ref):
    D = x_ref.shape[1]
    O = out_ref.shape[1]
    s = std_ref[0, 0].astype(jnp.float32)
    inv = 0.5 / (s * s)

    @pl.when(pl.program_id(0) == 0)
    def _prep():
        w = w_ref[:]
        w1, w2b, w3 = _split3(w)
        m2f = jnp.float32(-2.0)
        n2w1 = (w1.astype(jnp.float32) * m2f).astype(jnp.bfloat16)
        wc_ref[pl.ds(0, D), :] = n2w1
        wc_ref[pl.ds(D, D), :] = n2w1
        wc_ref[pl.ds(2 * D, D), :] = n2w1
        wsq1, wsq2, wsq3 = _split3(jnp.sum(w * w, axis=0, keepdims=True))
        wc_ref[pl.ds(3 * D, 1), :] = wsq1
        wc_ref[pl.ds(3 * D + 1, 1), :] = wsq2
        wc_ref[pl.ds(3 * D + 2, 1), :] = wsq3
        n2w2 = (w2b.astype(jnp.float32) * m2f).astype(jnp.bfloat16)
        wc_ref[pl.ds(3 * D + 3, D), :] = n2w2
        wc_ref[pl.ds(4 * D + 3, D), :] = (w3.astype(jnp.float32) * m2f
                                          ).astype(jnp.bfloat16)
        wc_ref[pl.ds(5 * D + 3, D), :] = n2w2
        oid = jax.lax.broadcasted_iota(jnp.int32, (1, O), 1)
        grf = (oid // side).astype(jnp.float32)
        gcf = (oid % side).astype(jnp.float32)
        gh, gl = _split2(-inv * (grf * grf + gcf * gcf))
        grb = grf.astype(jnp.bfloat16)
        gcb = gcf.astype(jnp.bfloat16)
        oneb = jnp.ones((1, O), dtype=jnp.bfloat16)
        wg_ref[pl.ds(0, 1), :] = grb
        wg_ref[pl.ds(1, 1), :] = grb
        wg_ref[pl.ds(2, 1), :] = gcb
        wg_ref[pl.ds(3, 1), :] = gcb
        wg_ref[pl.ds(4, 1), :] = oneb
        wg_ref[pl.ds(5, 1), :] = oneb
        wg_ref[pl.ds(6, 1), :] = gh
        wg_ref[pl.ds(7, 1), :] = gl

    x = x_ref[:]
    BB = x.shape[0]

    x1, x2b, x3 = _split3(x)
    ones3 = jnp.ones((BB, 3), dtype=jnp.bfloat16)
    xc = jnp.concatenate([x1, x2b, x3, ones3, x1, x1, x2b], axis=1)
    m2 = _dot(xc, wc_ref[:])
    x2 = jnp.sum(x * x, axis=1, keepdims=True)

    idx = jnp.argmin(m2, axis=1, keepdims=True)

    rowf = (idx // side).astype(jnp.float32)
    colf = (idx % side).astype(jnp.float32)
    t = jax.lax.broadcasted_iota(jnp.int32, (BB, side), 1).astype(jnp.float32)
    er = jnp.exp(-inv * (t - rowf) ** 2)
    ec = jnp.exp(-inv * (t - colf) ** 2)
    lrecip = -jnp.log(jnp.sum(er, axis=1, keepdims=True)
                      * jnp.sum(ec, axis=1, keepdims=True))
    u = (2.0 * inv) * rowf
    v = (2.0 * inv) * colf
    k = lrecip - inv * (rowf * rowf + colf * colf)
    uh, ul = _split2(u)
    vh, vl = _split2(v)
    kh, kl = _split2(k)
    one1 = jnp.ones((BB, 1), dtype=jnp.bfloat16)
    xg = jnp.concatenate([uh, ul, vh, vl, kh, kl, one1, one1], axis=1)
    arg = _dot(xg, wg_ref[:])
    out_ref[:] = (m2 + x2) * jnp.exp(arg)


def kernel(x, std, weights):
    B, D = x.shape
    O = weights.shape[1]
    side = int(round(float(O) ** 0.5))
    BB = 512 if B % 512 == 0 else B

    std2d = jnp.reshape(jnp.asarray(std), (1, 1))
    body = functools.partial(_tm_kernel, side)
    return pl.pallas_call(
        body,
        grid=(B // BB,),
        in_specs=[
            pl.BlockSpec(memory_space=pltpu.SMEM),
            pl.BlockSpec((BB, D), lambda i: (i, 0)),
            pl.BlockSpec((D, O), lambda i: (0, 0)),
        ],
        out_specs=pl.BlockSpec((BB, O), lambda i: (i, 0)),
        out_shape=jax.ShapeDtypeStruct((B, O), jnp.float32),
        scratch_shapes=[
            pltpu.VMEM((6 * D + 3, O), jnp.bfloat16),
            pltpu.VMEM((8, O), jnp.bfloat16),
        ],
    )(std2d, x, weights)
